# baseline (speedup 1.0000x reference)
import functools

import jax
import jax.numpy as jnp
from jax import lax
from jax.experimental import pallas as pl
from jax.experimental.pallas import tpu as pltpu
from jax.experimental.pallas import tpu_sc as plsc

D = 128
DE = 64
H = 128
NC = 2
NS = 16
NW = NC * NS


def _relu(v):
    return jnp.maximum(v, 0.0)


def _dot(a, b):
    return jax.lax.dot(a, b, preferred_element_type=jnp.float32)


def _build(n, e, interpret=False):
    assert e % 2 == 0
    e2 = e // 2
    npt = -(-n // (NS * 8)) * 8
    n_pad = npt * NS

    mesh = plsc.VectorSubcoreMesh(core_axis_name="c", subcore_axis_name="s",
                                  num_cores=NC, num_subcores=NS)

    def make_gather(e_off, eh):
        per_w = eh // NW
        gcb = 200 if per_w % 200 == 0 else per_w
        assert per_w % gcb == 0 and gcb % 8 == 0
        gnch = per_w // gcb
        assert gnch >= 3
        gsub = 40 if gcb % 40 == 0 else gcb
        assert gcb % gsub == 0 and gsub <= 128 and gsub % 8 == 0
        gnsub = gcb // gsub
        npair = (gnch - 2) // 2

        @functools.partial(
            pl.kernel,
            out_type=(jax.ShapeDtypeStruct((eh, D), jnp.float32),
                      jax.ShapeDtypeStruct((eh, D), jnp.float32)),
            mesh=mesh,
            scratch_types=(
                [pltpu.VMEM((gcb,), jnp.int32)] * 4 +
                [pltpu.VMEM((gcb, D), jnp.float32)] * 4 +
                [pltpu.SemaphoreType.DMA] * 4
            ),
            interpret=interpret,
        )
        def sc_gather2(ta, tb, src, dst, oa, ob, *scr):
            idx_a = scr[0:2]
            idx_b = scr[2:4]
            ra = scr[4:6]
            rb = scr[6:8]
            gsem = scr[8:10]
            wsem = scr[10:12]
            wid = lax.axis_index("s") * NC + lax.axis_index("c")
            lbase = pl.multiple_of(wid * per_w, 8)

            def load_idx(b, c):
                off = pl.multiple_of(e_off + lbase + c * gcb, 8)
                pltpu.sync_copy(src.at[pl.ds(off, gcb)], idx_a[b])
                pltpu.sync_copy(dst.at[pl.ds(off, gcb)], idx_b[b])

            def g_copies(b):
                res = []
                for j in range(gnsub):
                    s = j * gsub
                    res.append(pltpu.make_async_copy(
                        ta.at[idx_a[b].at[pl.ds(s, gsub)]],
                        ra[b].at[pl.ds(s, gsub)], gsem[b]))
                    res.append(pltpu.make_async_copy(
                        tb.at[idx_b[b].at[pl.ds(s, gsub)]],
                        rb[b].at[pl.ds(s, gsub)], gsem[b]))
                return res

            def start_g(b):
                for cp in g_copies(b):
                    cp.start()

            def wait_g(b):
                for cp in g_copies(b):
                    cp.wait()

            def w_copies(b, c):
                off = pl.multiple_of(lbase + c * gcb, 8)
                return [pltpu.make_async_copy(ra[b], oa.at[pl.ds(off, gcb)],
                                              wsem[b]),
                        pltpu.make_async_copy(rb[b], ob.at[pl.ds(off, gcb)],
                                              wsem[b])]

            def start_wb(b, c):
                for cp in w_copies(b, c):
                    cp.start()

            def wait_wb(b, c):
                for cp in w_copies(b, c):
                    cp.wait()

            load_idx(0, 0)
            start_g(0)
            wait_g(0)
            start_wb(0, 0)
            load_idx(1, 1)
            start_g(1)

            def step(b, c):
                wait_g(b)
                start_wb(b, c)
                wait_wb(1 - b, c - 1)
                load_idx(1 - b, c + 1)
                start_g(1 - b)

            def body(k, carry):
                step(1, 2 * k + 1)
                step(0, 2 * k + 2)
                return carry

            lax.fori_loop(0, npair, body, 0)
            for c in range(2 * npair + 1, gnch):
                b = c % 2
                wait_g(b)
                start_wb(b, c)
                wait_wb(1 - b, c - 1)
                if c + 1 < gnch:
                    load_idx(1 - b, c + 1)
                    start_g(1 - b)
            wait_wb((gnch - 1) % 2, gnch - 1)

        return sc_gather2

    def make_scatter(e_off, eh):
        per_w = eh // NW
        cbs = 80 if per_w % 80 == 0 else (40 if per_w % 40 == 0 else per_w)
        assert per_w % cbs == 0 and cbs % 8 == 0 and cbs <= 128
        nchs = per_w // cbs
        assert nchs >= 3
        npair = (nchs - 2) // 2
        pieces = [(o, min(cbs, npt - o)) for o in range(0, npt, cbs)]

        @functools.partial(
            pl.kernel,
            out_type=jax.ShapeDtypeStruct((NC * n_pad, D), jnp.float32),
            mesh=mesh,
            scratch_types=(
                [pltpu.VMEM((cbs,), jnp.int32)] * 2 +
                [pltpu.VMEM((cbs, D), jnp.float32)] * 2 +
                [pltpu.SemaphoreType.DMA] * 4 +
                [pltpu.VMEM_SHARED((n_pad, D), jnp.float32)]
            ),
            interpret=interpret,
        )
        def sc_scatter(vals, dst, zero, out, *scr):
            idxs = scr[0:2]
            rows = scr[2:4]
            lsem = scr[4:6]
            asem = scr[6:8]
            acc = scr[8]
            cid = lax.axis_index("c")
            sid = lax.axis_index("s")
            wid = sid * NC + cid
            lbase = pl.multiple_of(wid * per_w, 8)
            arow = pl.multiple_of(sid * npt, 8)

            pltpu.sync_copy(zero.at[pl.ds(0, cbs)], rows[0])
            for o, sz in pieces:
                pltpu.sync_copy(rows[0].at[pl.ds(0, sz)],
                                acc.at[pl.ds(arow + o, sz)])
            plsc.subcore_barrier()

            def l_copies(b, c):
                loff = pl.multiple_of(lbase + c * cbs, 8)
                goff = pl.multiple_of(e_off + lbase + c * cbs, 8)
                return [pltpu.make_async_copy(vals.at[pl.ds(loff, cbs)],
                                              rows[b], lsem[b]),
                        pltpu.make_async_copy(dst.at[pl.ds(goff, cbs)],
                                              idxs[b], lsem[b])]

            def start_load(b, c):
                for cp in l_copies(b, c):
                    cp.start()

            def wait_load(b, c):
                for cp in l_copies(b, c):
                    cp.wait()

            def a_copy(b):
                return pltpu.make_async_copy(rows[b], acc.at[idxs[b]],
                                             asem[b])

            start_load(0, 0)
            wait_load(0, 0)
            a_copy(0).start(add=True)
            start_load(1, 1)

            def step(b, c):
                wait_load(b, c)
                a_copy(b).start(add=True)
                a_copy(1 - b).wait()
                start_load(1 - b, c + 1)

            def body(k, carry):
                step(1, 2 * k + 1)
                step(0, 2 * k + 2)
                return carry

            lax.fori_loop(0, npair, body, 0)
            for c in range(2 * npair + 1, nchs):
                b = c % 2
                wait_load(b, c)
                a_copy(b).start(add=True)
                a_copy(1 - b).wait()
                if c + 1 < nchs:
                    start_load(1 - b, c + 1)
            a_copy((nchs - 1) % 2).wait()

            plsc.subcore_barrier()
            obase = pl.multiple_of(cid * n_pad + arow, 8)
            for o, sz in pieces:
                pltpu.sync_copy(acc.at[pl.ds(arow + o, sz)],
                                rows[0].at[pl.ds(0, sz)])
                pltpu.sync_copy(rows[0].at[pl.ds(0, sz)],
                                out.at[pl.ds(obase + o, sz)])

        return sc_scatter

    gather0 = make_gather(0, e2)
    gather1 = make_gather(e2, e2)
    scatter0 = make_scatter(0, e2)
    scatter1 = make_scatter(e2, e2)

    bn = 2000 if n % 2000 == 0 else n
    gn = n // bn
    be = 4000 if e2 % 4000 == 0 else e2
    ge2 = e2 // be

    def node_embed_body(x_ref, w_ref, b_ref, wab_ref, nf_ref, a_ref, b2_ref):
        nf = _relu(_dot(x_ref[...], w_ref[...]) + b_ref[...])
        nf_ref[...] = nf
        ab = _dot(nf, wab_ref[...])
        a_ref[...] = ab[:, :D]
        b2_ref[...] = ab[:, D:]

    node_embed = pl.pallas_call(
        node_embed_body,
        grid=(gn,),
        in_specs=[
            pl.BlockSpec((bn, D), lambda i: (i, 0)),
            pl.BlockSpec((D, D), lambda i: (0, 0)),
            pl.BlockSpec((1, D), lambda i: (0, 0)),
            pl.BlockSpec((D, 2 * D), lambda i: (0, 0)),
        ],
        out_specs=[
            pl.BlockSpec((bn, D), lambda i: (i, 0)),
            pl.BlockSpec((bn, D), lambda i: (i, 0)),
            pl.BlockSpec((bn, D), lambda i: (i, 0)),
        ],
        out_shape=[jax.ShapeDtypeStruct((n, D), jnp.float32)] * 3,
        interpret=interpret,
    )

    def edge_embed_body(ea_ref, w_ref, b_ref, ef_ref):
        ef = _relu(_dot(ea_ref[...], w_ref[...]) + b_ref[...])
        ef_ref[...] = jnp.concatenate(
            [ef, jnp.zeros((ef.shape[0], D - DE), jnp.float32)], axis=1)

    def make_edge_embed(blk_off):
        return pl.pallas_call(
            edge_embed_body,
            grid=(ge2,),
            in_specs=[
                pl.BlockSpec((be, 16), lambda i: (i + blk_off, 0)),
                pl.BlockSpec((16, DE), lambda i: (0, 0)),
                pl.BlockSpec((1, DE), lambda i: (0, 0)),
            ],
            out_specs=pl.BlockSpec((be, D), lambda i: (i, 0)),
            out_shape=jax.ShapeDtypeStruct((e2, D), jnp.float32),
            interpret=interpret,
        )

    edge_embed0 = make_edge_embed(0)
    edge_embed1 = make_edge_embed(ge2)

    def edge_mlp_body(sa_ref, sb_ref, ef_ref, wc_ref, b1_ref, w2_ref, b2_ref,
                      out_ref):
        h = _relu(sa_ref[...] + sb_ref[...]
                  + _dot(ef_ref[:, :DE], wc_ref[...]) + b1_ref[...])
        ef2 = _relu(_dot(h, w2_ref[...]) + b2_ref[...])
        out_ref[...] = jnp.concatenate(
            [ef2, jnp.zeros((ef2.shape[0], D - DE), jnp.float32)], axis=1)

    edge_mlp = pl.pallas_call(
        edge_mlp_body,
        grid=(ge2,),
        in_specs=[
            pl.BlockSpec((be, D), lambda i: (i, 0)),
            pl.BlockSpec((be, D), lambda i: (i, 0)),
            pl.BlockSpec((be, D), lambda i: (i, 0)),
            pl.BlockSpec((DE, H), lambda i: (0, 0)),
            pl.BlockSpec((1, H), lambda i: (0, 0)),
            pl.BlockSpec((H, DE), lambda i: (0, 0)),
            pl.BlockSpec((1, DE), lambda i: (0, 0)),
        ],
        out_specs=pl.BlockSpec((be, D), lambda i: (i, 0)),
        out_shape=jax.ShapeDtypeStruct((e2, D), jnp.float32),
        interpret=interpret,
    )

    def node_update_body(nf_ref, g00_ref, g01_ref, g10_ref, g11_ref,
                         wa_ref, wb_ref, b_ref, wab_ref,
                         nf2_ref, a_ref, b2_ref):
        agg = (g00_ref[:, :DE] + g01_ref[:, :DE]
               + g10_ref[:, :DE] + g11_ref[:, :DE])
        nf2 = _relu(_dot(nf_ref[...], wa_ref[...]) + _dot(agg, wb_ref[...])
                    + b_ref[...])
        nf2_ref[...] = nf2
        ab = _dot(nf2, wab_ref[...])
        a_ref[...] = ab[:, :D]
        b2_ref[...] = ab[:, D:]

    node_update = pl.pallas_call(
        node_update_body,
        grid=(gn,),
        in_specs=[
            pl.BlockSpec((bn, D), lambda i: (i, 0)),
            pl.BlockSpec((bn, D), lambda i: (i, 0)),
            pl.BlockSpec((bn, D), lambda i: (i, 0)),
            pl.BlockSpec((bn, D), lambda i: (i, 0)),
            pl.BlockSpec((bn, D), lambda i: (i, 0)),
            pl.BlockSpec((D, D), lambda i: (0, 0)),
            pl.BlockSpec((DE, D), lambda i: (0, 0)),
            pl.BlockSpec((1, D), lambda i: (0, 0)),
            pl.BlockSpec((D, 2 * D), lambda i: (0, 0)),
        ],
        out_specs=[
            pl.BlockSpec((bn, D), lambda i: (i, 0)),
            pl.BlockSpec((bn, D), lambda i: (i, 0)),
            pl.BlockSpec((bn, D), lambda i: (i, 0)),
        ],
        out_shape=[jax.ShapeDtypeStruct((n, D), jnp.float32)] * 3,
        interpret=interpret,
    )

    def node_final_body(nf_ref, g00_ref, g01_ref, g10_ref, g11_ref,
                        wa_ref, wb_ref, b_ref,
                        wt1_ref, bt1_ref, wt2_ref, bt2_ref, p_ref):
        agg = (g00_ref[:, :DE] + g01_ref[:, :DE]
               + g10_ref[:, :DE] + g11_ref[:, :DE])
        nf2 = _relu(_dot(nf_ref[...], wa_ref[...]) + _dot(agg, wb_ref[...])
                    + b_ref[...])
        t = _relu(_dot(nf2, wt1_ref[...]) + bt1_ref[...])
        p_ref[...] = _dot(t, wt2_ref[...]) + bt2_ref[...]

    node_final = pl.pallas_call(
        node_final_body,
        grid=(gn,),
        in_specs=[
            pl.BlockSpec((bn, D), lambda i: (i, 0)),
            pl.BlockSpec((bn, D), lambda i: (i, 0)),
            pl.BlockSpec((bn, D), lambda i: (i, 0)),
            pl.BlockSpec((bn, D), lambda i: (i, 0)),
            pl.BlockSpec((bn, D), lambda i: (i, 0)),
            pl.BlockSpec((D, D), lambda i: (0, 0)),
            pl.BlockSpec((DE, D), lambda i: (0, 0)),
            pl.BlockSpec((1, D), lambda i: (0, 0)),
            pl.BlockSpec((D, DE), lambda i: (0, 0)),
            pl.BlockSpec((1, DE), lambda i: (0, 0)),
            pl.BlockSpec((DE, 1), lambda i: (0, 0)),
            pl.BlockSpec((1, 1), lambda i: (0, 0)),
        ],
        out_specs=pl.BlockSpec((bn, 1), lambda i: (i, 0)),
        out_shape=jax.ShapeDtypeStruct((n, 1), jnp.float32),
        interpret=interpret,
    )

    def run(x, edge_attr, edge_index, Wn0, bn0, We0, be0, Wm1, bm1, Wm2, bm2,
            Wu, bu, Wt1, bt1, Wt2, bt2):
        src = edge_index[0].astype(jnp.int32)
        dst = edge_index[1].astype(jnp.int32)
        WmA = Wm1[:D]
        WmB = Wm1[D:2 * D]
        WmC = Wm1[2 * D:]
        WAB = jnp.concatenate([WmA, WmB], axis=1)
        WuA = Wu[:D]
        WuB = Wu[D:]
        zero = jnp.zeros((n_pad, D), jnp.float32)
        b1r = bm1.reshape(1, -1)
        b2r = bm2.reshape(1, -1)
        bur = bu.reshape(1, -1)

        nf, A, B = node_embed(x, Wn0, bn0.reshape(1, -1), WAB)
        ef0 = edge_embed0(edge_attr, We0, be0.reshape(1, -1))
        ef1 = edge_embed1(edge_attr, We0, be0.reshape(1, -1))

        def mp_step(A, B, ef0, ef1):
            sa0, sb0 = gather0(A, B, src, dst)
            sa1, sb1 = gather1(A, B, src, dst)
            ef0n = edge_mlp(sa0, sb0, ef0, WmC, b1r, Wm2, b2r)
            ef1n = edge_mlp(sa1, sb1, ef1, WmC, b1r, Wm2, b2r)
            p0 = scatter0(ef0n, dst, zero)
            p1 = scatter1(ef1n, dst, zero)
            return ef0n, ef1n, p0, p1

        ef0, ef1, p0, p1 = mp_step(A, B, ef0, ef1)
        nf, A, B = node_update(nf, p0[:n], p0[n_pad:n_pad + n],
                               p1[:n], p1[n_pad:n_pad + n],
                               WuA, WuB, bur, WAB)
        ef0, ef1, p0, p1 = mp_step(A, B, ef0, ef1)
        preds = node_final(nf, p0[:n], p0[n_pad:n_pad + n],
                           p1[:n], p1[n_pad:n_pad + n],
                           WuA, WuB, bur,
                           Wt1, bt1.reshape(1, -1), Wt2, bt2.reshape(1, -1))
        return preds.squeeze(-1)

    return run


@functools.cache
def _pipeline():
    return _build(10000, 320000)


def kernel(x, edge_attr, edge_index, Wn0, bn0, We0, be0, Wm1, bm1, Wm2, bm2,
           Wu, bu, Wt1, bt1, Wt2, bt2):
    return _pipeline()(x, edge_attr, edge_index, Wn0, bn0, We0, be0,
                       Wm1, bm1, Wm2, bm2, Wu, bu, Wt1, bt1, Wt2, bt2)


# speedup vs baseline: 1.1241x; 1.1241x over previous
import functools

import jax
import jax.numpy as jnp
from jax import lax
from jax.experimental import pallas as pl
from jax.experimental.pallas import tpu as pltpu
from jax.experimental.pallas import tpu_sc as plsc

D = 128
DE = 64
H = 128
NC = 2
NS = 16
NW = NC * NS


def _relu(v):
    return jnp.maximum(v, 0.0)


def _dot(a, b):
    return jax.lax.dot(a, b, preferred_element_type=jnp.float32)


def _build(n, e, interpret=False):
    assert e % 2 == 0
    e2 = e // 2
    npt = -(-n // (NS * 8)) * 8
    n_pad = npt * NS

    mesh = plsc.VectorSubcoreMesh(core_axis_name="c", subcore_axis_name="s",
                                  num_cores=NC, num_subcores=NS)

    def make_gather(e_off, eh):
        per_w = eh // NW
        gcb = 200 if per_w % 200 == 0 else per_w
        assert per_w % gcb == 0 and gcb % 8 == 0
        gnch = per_w // gcb
        assert gnch >= 3
        gsub = 40 if gcb % 40 == 0 else gcb
        assert gcb % gsub == 0 and gsub <= 128 and gsub % 8 == 0
        gnsub = gcb // gsub
        npair = (gnch - 2) // 2

        @functools.partial(
            pl.kernel,
            out_type=(jax.ShapeDtypeStruct((eh, D), jnp.float32),
                      jax.ShapeDtypeStruct((eh, D), jnp.float32)),
            mesh=mesh,
            scratch_types=(
                [pltpu.VMEM((gcb,), jnp.int32)] * 4 +
                [pltpu.VMEM((gcb, D), jnp.float32)] * 4 +
                [pltpu.SemaphoreType.DMA] * 4
            ),
            interpret=interpret,
        )
        def sc_gather2(ta, tb, src, dst, oa, ob, *scr):
            idx_a = scr[0:2]
            idx_b = scr[2:4]
            ra = scr[4:6]
            rb = scr[6:8]
            gsem = scr[8:10]
            wsem = scr[10:12]
            wid = lax.axis_index("s") * NC + lax.axis_index("c")
            lbase = pl.multiple_of(wid * per_w, 8)

            def load_idx(b, c):
                off = pl.multiple_of(e_off + lbase + c * gcb, 8)
                pltpu.sync_copy(src.at[pl.ds(off, gcb)], idx_a[b])
                pltpu.sync_copy(dst.at[pl.ds(off, gcb)], idx_b[b])

            def g_copies(b):
                res = []
                for j in range(gnsub):
                    s = j * gsub
                    res.append(pltpu.make_async_copy(
                        ta.at[idx_a[b].at[pl.ds(s, gsub)]],
                        ra[b].at[pl.ds(s, gsub)], gsem[b]))
                    res.append(pltpu.make_async_copy(
                        tb.at[idx_b[b].at[pl.ds(s, gsub)]],
                        rb[b].at[pl.ds(s, gsub)], gsem[b]))
                return res

            def start_g(b):
                for cp in g_copies(b):
                    cp.start()

            def wait_g(b):
                for cp in g_copies(b):
                    cp.wait()

            def w_copies(b, c):
                off = pl.multiple_of(lbase + c * gcb, 8)
                return [pltpu.make_async_copy(ra[b], oa.at[pl.ds(off, gcb)],
                                              wsem[b]),
                        pltpu.make_async_copy(rb[b], ob.at[pl.ds(off, gcb)],
                                              wsem[b])]

            def start_wb(b, c):
                for cp in w_copies(b, c):
                    cp.start()

            def wait_wb(b, c):
                for cp in w_copies(b, c):
                    cp.wait()

            load_idx(0, 0)
            start_g(0)
            wait_g(0)
            start_wb(0, 0)
            load_idx(1, 1)
            start_g(1)

            def step(b, c):
                wait_g(b)
                start_wb(b, c)
                wait_wb(1 - b, c - 1)
                load_idx(1 - b, c + 1)
                start_g(1 - b)

            def body(k, carry):
                step(1, 2 * k + 1)
                step(0, 2 * k + 2)
                return carry

            lax.fori_loop(0, npair, body, 0)
            for c in range(2 * npair + 1, gnch):
                b = c % 2
                wait_g(b)
                start_wb(b, c)
                wait_wb(1 - b, c - 1)
                if c + 1 < gnch:
                    load_idx(1 - b, c + 1)
                    start_g(1 - b)
            wait_wb((gnch - 1) % 2, gnch - 1)

        return sc_gather2

    def make_scatter(e_off, eh):
        per_w = eh // NW
        cbs = 80 if per_w % 80 == 0 else (40 if per_w % 40 == 0 else per_w)
        assert per_w % cbs == 0 and cbs % 8 == 0 and cbs <= 128
        nchs = per_w // cbs
        assert nchs >= 3
        npair = (nchs - 2) // 2
        pieces = [(o, min(cbs, npt - o)) for o in range(0, npt, cbs)]

        @functools.partial(
            pl.kernel,
            out_type=jax.ShapeDtypeStruct((NC * n_pad, D), jnp.float32),
            mesh=mesh,
            scratch_types=(
                [pltpu.VMEM((cbs,), jnp.int32)] * 2 +
                [pltpu.VMEM((cbs, D), jnp.float32)] * 2 +
                [pltpu.SemaphoreType.DMA] * 4 +
                [pltpu.VMEM_SHARED((n_pad, D), jnp.float32)]
            ),
            interpret=interpret,
        )
        def sc_scatter(vals, dst, zero, out, *scr):
            idxs = scr[0:2]
            rows = scr[2:4]
            lsem = scr[4:6]
            asem = scr[6:8]
            acc = scr[8]
            cid = lax.axis_index("c")
            sid = lax.axis_index("s")
            wid = sid * NC + cid
            lbase = pl.multiple_of(wid * per_w, 8)
            arow = pl.multiple_of(sid * npt, 8)

            pltpu.sync_copy(zero.at[pl.ds(0, cbs)], rows[0])
            for o, sz in pieces:
                pltpu.sync_copy(rows[0].at[pl.ds(0, sz)],
                                acc.at[pl.ds(arow + o, sz)])
            plsc.subcore_barrier()

            def l_copies(b, c):
                loff = pl.multiple_of(lbase + c * cbs, 8)
                goff = pl.multiple_of(e_off + lbase + c * cbs, 8)
                return [pltpu.make_async_copy(vals.at[pl.ds(loff, cbs)],
                                              rows[b], lsem[b]),
                        pltpu.make_async_copy(dst.at[pl.ds(goff, cbs)],
                                              idxs[b], lsem[b])]

            def start_load(b, c):
                for cp in l_copies(b, c):
                    cp.start()

            def wait_load(b, c):
                for cp in l_copies(b, c):
                    cp.wait()

            def a_copy(b):
                return pltpu.make_async_copy(rows[b], acc.at[idxs[b]],
                                             asem[b])

            start_load(0, 0)
            wait_load(0, 0)
            a_copy(0).start(add=True)
            start_load(1, 1)

            def step(b, c):
                wait_load(b, c)
                a_copy(b).start(add=True)
                a_copy(1 - b).wait()
                start_load(1 - b, c + 1)

            def body(k, carry):
                step(1, 2 * k + 1)
                step(0, 2 * k + 2)
                return carry

            lax.fori_loop(0, npair, body, 0)
            for c in range(2 * npair + 1, nchs):
                b = c % 2
                wait_load(b, c)
                a_copy(b).start(add=True)
                a_copy(1 - b).wait()
                if c + 1 < nchs:
                    start_load(1 - b, c + 1)
            a_copy((nchs - 1) % 2).wait()

            plsc.subcore_barrier()
            obase = pl.multiple_of(cid * n_pad + arow, 8)
            for o, sz in pieces:
                pltpu.sync_copy(acc.at[pl.ds(arow + o, sz)],
                                rows[0].at[pl.ds(0, sz)])
                pltpu.sync_copy(rows[0].at[pl.ds(0, sz)],
                                out.at[pl.ds(obase + o, sz)])

        return sc_scatter

    gather = make_gather(0, e)
    scatter = make_scatter(0, e)

    bn = 2000 if n % 2000 == 0 else n
    gn = n // bn
    be = 4000 if e % 4000 == 0 else e
    ge = e // be

    def node_embed_body(x_ref, w_ref, b_ref, wab_ref, nf_ref, a_ref, b2_ref):
        nf = _relu(_dot(x_ref[...], w_ref[...]) + b_ref[...])
        nf_ref[...] = nf
        ab = _dot(nf, wab_ref[...])
        a_ref[...] = ab[:, :D]
        b2_ref[...] = ab[:, D:]

    node_embed = pl.pallas_call(
        node_embed_body,
        grid=(gn,),
        in_specs=[
            pl.BlockSpec((bn, D), lambda i: (i, 0)),
            pl.BlockSpec((D, D), lambda i: (0, 0)),
            pl.BlockSpec((1, D), lambda i: (0, 0)),
            pl.BlockSpec((D, 2 * D), lambda i: (0, 0)),
        ],
        out_specs=[
            pl.BlockSpec((bn, D), lambda i: (i, 0)),
            pl.BlockSpec((bn, D), lambda i: (i, 0)),
            pl.BlockSpec((bn, D), lambda i: (i, 0)),
        ],
        out_shape=[jax.ShapeDtypeStruct((n, D), jnp.float32)] * 3,
        interpret=interpret,
    )

    def edge_mlp1_body(sa_ref, sb_ref, ea_ref, we_ref, be_ref, wc_ref,
                       b1_ref, w2_ref, b2_ref, out_ref):
        ef = _relu(_dot(ea_ref[...], we_ref[...]) + be_ref[...])
        h = _relu(sa_ref[...] + sb_ref[...] + _dot(ef, wc_ref[...])
                  + b1_ref[...])
        ef2 = _relu(_dot(h, w2_ref[...]) + b2_ref[...])
        out_ref[...] = jnp.concatenate(
            [ef2, jnp.zeros((ef2.shape[0], D - DE), jnp.float32)], axis=1)

    edge_mlp1 = pl.pallas_call(
        edge_mlp1_body,
        grid=(ge,),
        in_specs=[
            pl.BlockSpec((be, D), lambda i: (i, 0)),
            pl.BlockSpec((be, D), lambda i: (i, 0)),
            pl.BlockSpec((be, 16), lambda i: (i, 0)),
            pl.BlockSpec((16, DE), lambda i: (0, 0)),
            pl.BlockSpec((1, DE), lambda i: (0, 0)),
            pl.BlockSpec((DE, H), lambda i: (0, 0)),
            pl.BlockSpec((1, H), lambda i: (0, 0)),
            pl.BlockSpec((H, DE), lambda i: (0, 0)),
            pl.BlockSpec((1, DE), lambda i: (0, 0)),
        ],
        out_specs=pl.BlockSpec((be, D), lambda i: (i, 0)),
        out_shape=jax.ShapeDtypeStruct((e, D), jnp.float32),
        interpret=interpret,
    )

    def edge_mlp_body(sa_ref, sb_ref, ef_ref, wc_ref, b1_ref, w2_ref, b2_ref,
                      out_ref):
        h = _relu(sa_ref[...] + sb_ref[...]
                  + _dot(ef_ref[:, :DE], wc_ref[...]) + b1_ref[...])
        ef2 = _relu(_dot(h, w2_ref[...]) + b2_ref[...])
        out_ref[...] = jnp.concatenate(
            [ef2, jnp.zeros((ef2.shape[0], D - DE), jnp.float32)], axis=1)

    edge_mlp = pl.pallas_call(
        edge_mlp_body,
        grid=(ge,),
        in_specs=[
            pl.BlockSpec((be, D), lambda i: (i, 0)),
            pl.BlockSpec((be, D), lambda i: (i, 0)),
            pl.BlockSpec((be, D), lambda i: (i, 0)),
            pl.BlockSpec((DE, H), lambda i: (0, 0)),
            pl.BlockSpec((1, H), lambda i: (0, 0)),
            pl.BlockSpec((H, DE), lambda i: (0, 0)),
            pl.BlockSpec((1, DE), lambda i: (0, 0)),
        ],
        out_specs=pl.BlockSpec((be, D), lambda i: (i, 0)),
        out_shape=jax.ShapeDtypeStruct((e, D), jnp.float32),
        interpret=interpret,
    )

    def node_update_body(nf_ref, g0_ref, g1_ref,
                         wa_ref, wb_ref, b_ref, wab_ref,
                         nf2_ref, a_ref, b2_ref):
        agg = g0_ref[:, :DE] + g1_ref[:, :DE]
        nf2 = _relu(_dot(nf_ref[...], wa_ref[...]) + _dot(agg, wb_ref[...])
                    + b_ref[...])
        nf2_ref[...] = nf2
        ab = _dot(nf2, wab_ref[...])
        a_ref[...] = ab[:, :D]
        b2_ref[...] = ab[:, D:]

    node_update = pl.pallas_call(
        node_update_body,
        grid=(gn,),
        in_specs=[
            pl.BlockSpec((bn, D), lambda i: (i, 0)),
            pl.BlockSpec((bn, D), lambda i: (i, 0)),
            pl.BlockSpec((bn, D), lambda i: (i, 0)),
            pl.BlockSpec((D, D), lambda i: (0, 0)),
            pl.BlockSpec((DE, D), lambda i: (0, 0)),
            pl.BlockSpec((1, D), lambda i: (0, 0)),
            pl.BlockSpec((D, 2 * D), lambda i: (0, 0)),
        ],
        out_specs=[
            pl.BlockSpec((bn, D), lambda i: (i, 0)),
            pl.BlockSpec((bn, D), lambda i: (i, 0)),
            pl.BlockSpec((bn, D), lambda i: (i, 0)),
        ],
        out_shape=[jax.ShapeDtypeStruct((n, D), jnp.float32)] * 3,
        interpret=interpret,
    )

    def node_final_body(nf_ref, g0_ref, g1_ref,
                        wa_ref, wb_ref, b_ref,
                        wt1_ref, bt1_ref, wt2_ref, bt2_ref, p_ref):
        agg = g0_ref[:, :DE] + g1_ref[:, :DE]
        nf2 = _relu(_dot(nf_ref[...], wa_ref[...]) + _dot(agg, wb_ref[...])
                    + b_ref[...])
        t = _relu(_dot(nf2, wt1_ref[...]) + bt1_ref[...])
        p_ref[...] = _dot(t, wt2_ref[...]) + bt2_ref[...]

    node_final = pl.pallas_call(
        node_final_body,
        grid=(gn,),
        in_specs=[
            pl.BlockSpec((bn, D), lambda i: (i, 0)),
            pl.BlockSpec((bn, D), lambda i: (i, 0)),
            pl.BlockSpec((bn, D), lambda i: (i, 0)),
            pl.BlockSpec((D, D), lambda i: (0, 0)),
            pl.BlockSpec((DE, D), lambda i: (0, 0)),
            pl.BlockSpec((1, D), lambda i: (0, 0)),
            pl.BlockSpec((D, DE), lambda i: (0, 0)),
            pl.BlockSpec((1, DE), lambda i: (0, 0)),
            pl.BlockSpec((DE, 1), lambda i: (0, 0)),
            pl.BlockSpec((1, 1), lambda i: (0, 0)),
        ],
        out_specs=pl.BlockSpec((bn, 1), lambda i: (i, 0)),
        out_shape=jax.ShapeDtypeStruct((n, 1), jnp.float32),
        interpret=interpret,
    )

    def run(x, edge_attr, edge_index, Wn0, bn0, We0, be0, Wm1, bm1, Wm2, bm2,
            Wu, bu, Wt1, bt1, Wt2, bt2):
        src = edge_index[0].astype(jnp.int32)
        dst = edge_index[1].astype(jnp.int32)
        WmA = Wm1[:D]
        WmB = Wm1[D:2 * D]
        WmC = Wm1[2 * D:]
        WAB = jnp.concatenate([WmA, WmB], axis=1)
        WuA = Wu[:D]
        WuB = Wu[D:]
        zero = jnp.zeros((n_pad, D), jnp.float32)
        b1r = bm1.reshape(1, -1)
        b2r = bm2.reshape(1, -1)
        bur = bu.reshape(1, -1)

        nf, A, B = node_embed(x, Wn0, bn0.reshape(1, -1), WAB)

        sa, sb = gather(A, B, src, dst)
        ef = edge_mlp1(sa, sb, edge_attr, We0, be0.reshape(1, -1),
                       WmC, b1r, Wm2, b2r)
        p = scatter(ef, dst, zero)
        nf, A, B = node_update(nf, p[:n], p[n_pad:n_pad + n],
                               WuA, WuB, bur, WAB)
        sa, sb = gather(A, B, src, dst)
        ef = edge_mlp(sa, sb, ef, WmC, b1r, Wm2, b2r)
        p = scatter(ef, dst, zero)
        preds = node_final(nf, p[:n], p[n_pad:n_pad + n],
                           WuA, WuB, bur,
                           Wt1, bt1.reshape(1, -1), Wt2, bt2.reshape(1, -1))
        return preds.squeeze(-1)

    return run


@functools.cache
def _pipeline():
    return _build(10000, 320000)


def kernel(x, edge_attr, edge_index, Wn0, bn0, We0, be0, Wm1, bm1, Wm2, bm2,
           Wu, bu, Wt1, bt1, Wt2, bt2):
    return _pipeline()(x, edge_attr, edge_index, Wn0, bn0, We0, be0,
                       Wm1, bm1, Wm2, bm2, Wu, bu, Wt1, bt1, Wt2, bt2)
